# baseline (speedup 1.0000x reference)
import functools

import jax
import jax.numpy as jnp
from jax import lax
from jax.experimental import pallas as pl
from jax.experimental.pallas import tpu as pltpu
from jax.experimental.pallas import tpu_sc as plsc

B = 4096
DIM = 1024
NUM_EXPERTS = 8
NUM_HEADS = 16
TOP_K = 2
HEAD_DIM = DIM // NUM_HEADS
SCALE = HEAD_DIM ** (-0.5)

BM = 256
NPAIR = TOP_K * B
PADDED = NPAIR + NUM_EXPERTS * BM
NBLOCKS = PADDED // BM
NW = 32
TPW = B // NW
CB = 16
BT_ATTN = 512


def _router_kernel(x_ref, wg_ref, bg_ref,
                   pos0_ref, pos1_ref, w1_ref, w2_ref, eb_ref):
    scores = jnp.dot(x_ref[...], wg_ref[...],
                     preferred_element_type=jnp.float32) + bg_ref[...]
    scores = scores - jnp.max(scores, axis=1, keepdims=True)
    scores = jnp.exp(scores)
    gates = scores / jnp.sum(scores, axis=1, keepdims=True)

    e_iota = lax.broadcasted_iota(jnp.int32, (B, NUM_EXPERTS), 1)
    m1 = jnp.max(gates, axis=1, keepdims=True)
    idx1 = jnp.min(jnp.where(gates == m1, e_iota, NUM_EXPERTS), axis=1,
                   keepdims=True)
    sel1 = e_iota == idx1
    masked = jnp.where(sel1, -1.0, gates)
    m2 = jnp.max(masked, axis=1, keepdims=True)
    idx2 = jnp.min(jnp.where(masked == m2, e_iota, NUM_EXPERTS), axis=1,
                   keepdims=True)
    sel2 = e_iota == idx2

    occ = (sel1 | sel2).astype(jnp.float32)
    sub = 512
    r_io = lax.broadcasted_iota(jnp.int32, (sub, sub), 0)
    c_io = lax.broadcasted_iota(jnp.int32, (sub, sub), 1)
    tri = (c_io < r_io).astype(jnp.float32)
    prefix = jnp.zeros((1, NUM_EXPERTS), jnp.float32)
    parts = []
    for blk in range(B // sub):
        occ_b = occ[blk * sub:(blk + 1) * sub, :]
        parts.append(jnp.dot(tri, occ_b,
                             preferred_element_type=jnp.float32) + prefix)
        prefix = prefix + jnp.sum(occ_b, axis=0, keepdims=True)
    cnt = jnp.concatenate(parts, axis=0)
    counts = prefix

    pcount = jnp.ceil(counts / BM) * BM
    u_io8 = lax.broadcasted_iota(jnp.int32, (NUM_EXPERTS, NUM_EXPERTS), 0)
    v_io8 = lax.broadcasted_iota(jnp.int32, (NUM_EXPERTS, NUM_EXPERTS), 1)
    tri8 = (u_io8 < v_io8).astype(jnp.float32)
    offs = jnp.dot(pcount, tri8,
                   preferred_element_type=jnp.float32)
    pend = offs + pcount

    sel1f = sel1.astype(jnp.float32)
    sel2f = sel2.astype(jnp.float32)
    pos0 = jnp.sum(sel1f * (offs + cnt), axis=1, keepdims=True)
    pos1 = jnp.sum(sel2f * (offs + cnt), axis=1, keepdims=True)
    pos0_ref[...] = pos0.astype(jnp.int32)
    pos1_ref[...] = pos1.astype(jnp.int32)
    w1_ref[...] = jnp.broadcast_to(m1, (B, 16))
    w2_ref[...] = jnp.broadcast_to(m2, (B, 16))

    lane8 = lax.broadcasted_iota(jnp.int32, (1, NUM_EXPERTS), 1)
    b_io = (lax.broadcasted_iota(jnp.int32, (1, 128), 1) * BM).astype(jnp.float32)
    ebv = jnp.zeros((1, 128), jnp.float32)
    for e in range(NUM_EXPERTS):
        pe = jnp.sum(jnp.where(lane8 == e, pend, 0.0), axis=1, keepdims=True)
        ebv = ebv + jnp.where(pe <= b_io, 1.0, 0.0)
    eb_ref[...] = jnp.minimum(ebv, NUM_EXPERTS - 1).astype(jnp.int32)


def _run_router(x, W_gate, b_gate):
    return pl.pallas_call(
        _router_kernel,
        grid=(1,),
        in_specs=[
            pl.BlockSpec((B, DIM), lambda i: (0, 0)),
            pl.BlockSpec((DIM, NUM_EXPERTS), lambda i: (0, 0)),
            pl.BlockSpec((1, NUM_EXPERTS), lambda i: (0, 0)),
        ],
        out_specs=[
            pl.BlockSpec((B, 1), lambda i: (0, 0)),
            pl.BlockSpec((B, 1), lambda i: (0, 0)),
            pl.BlockSpec((B, 16), lambda i: (0, 0)),
            pl.BlockSpec((B, 16), lambda i: (0, 0)),
            pl.BlockSpec((1, 128), lambda i: (0, 0)),
        ],
        out_shape=[
            jax.ShapeDtypeStruct((B, 1), jnp.int32),
            jax.ShapeDtypeStruct((B, 1), jnp.int32),
            jax.ShapeDtypeStruct((B, 16), jnp.float32),
            jax.ShapeDtypeStruct((B, 16), jnp.float32),
            jax.ShapeDtypeStruct((1, 128), jnp.int32),
        ],
    )(x, W_gate, b_gate.reshape(1, NUM_EXPERTS))


def _dispatch_sc(x, y, pos0, pos1):
    mesh = plsc.VectorSubcoreMesh(core_axis_name="c", subcore_axis_name="s")
    hb = TPW // 2

    @functools.partial(
        pl.kernel, mesh=mesh,
        out_type=[
            jax.ShapeDtypeStruct((PADDED, DIM), jnp.float32),
            jax.ShapeDtypeStruct((PADDED, DIM), jnp.float32),
        ],
        scratch_types=[
            pltpu.VMEM((hb, DIM), jnp.float32),
            pltpu.VMEM((hb,), jnp.int32),
            pltpu.VMEM((hb,), jnp.int32),
            pltpu.SemaphoreType.DMA,
        ],
    )
    def k2(x_hbm, y_hbm, p0_hbm, p1_hbm, xs_hbm, ys_hbm,
           rows_v, i0_v, i1_v, sem):
        wid = lax.axis_index("s") * 2 + lax.axis_index("c")
        for half in range(2):
            base = wid * TPW + half * hb
            pltpu.sync_copy(p0_hbm.at[wid, half], i0_v)
            pltpu.sync_copy(p1_hbm.at[wid, half], i1_v)
            for src_hbm, dst_hbm in ((x_hbm, xs_hbm), (y_hbm, ys_hbm)):
                pltpu.sync_copy(src_hbm.at[pl.ds(base, hb)], rows_v)
                pltpu.async_copy(rows_v, dst_hbm.at[i0_v], sem).wait()
                pltpu.async_copy(rows_v, dst_hbm.at[i1_v], sem).wait()

    return k2(x, y, pos0, pos1)


def _gemm_kernel(eb_ref, xs_ref, ys_ref, w_ref, out_ref):
    c = pl.program_id(0)
    w = w_ref[0]

    @pl.when(c == 0)
    def _():
        out_ref[...] = jnp.dot(ys_ref[...], w,
                               preferred_element_type=jnp.float32)

    @pl.when(c > 0)
    def _():
        out_ref[...] = jnp.dot(xs_ref[...], w,
                               preferred_element_type=jnp.float32)


def _run_gemm(xs, ys, W_qkv, eb):
    grid_spec = pltpu.PrefetchScalarGridSpec(
        num_scalar_prefetch=1,
        grid=(3, NBLOCKS),
        in_specs=[
            pl.BlockSpec((BM, DIM), lambda c, b, eb: (b, 0)),
            pl.BlockSpec((BM, DIM), lambda c, b, eb: (b, 0)),
            pl.BlockSpec((1, DIM, DIM), lambda c, b, eb: (eb[b], 0, c)),
        ],
        out_specs=pl.BlockSpec((BM, DIM), lambda c, b, eb: (b, c)),
    )
    return pl.pallas_call(
        _gemm_kernel,
        grid_spec=grid_spec,
        out_shape=jax.ShapeDtypeStruct((PADDED, 3 * DIM), jnp.float32),
        compiler_params=pltpu.CompilerParams(
            dimension_semantics=("arbitrary", "arbitrary"),
        ),
    )(eb, xs, ys, W_qkv)


def _combine_sc(qkvs, pos0, pos1, w1, w2):
    mesh = plsc.VectorSubcoreMesh(core_axis_name="c", subcore_axis_name="s")
    rounds = TPW // CB
    nmaj = 3 * DIM // 128
    nsub = 128 // 16

    @functools.partial(
        pl.kernel, mesh=mesh,
        out_type=jax.ShapeDtypeStruct((B, nmaj, 128), jnp.float32),
        scratch_types=[
            pltpu.VMEM((CB, nmaj, 128), jnp.float32),
            pltpu.VMEM((CB, nmaj, 128), jnp.float32),
            pltpu.VMEM((CB,), jnp.int32),
            pltpu.VMEM((CB,), jnp.int32),
            pltpu.VMEM((CB, 1, 16), jnp.float32),
            pltpu.VMEM((CB, 1, 16), jnp.float32),
            pltpu.SemaphoreType.DMA,
        ],
    )
    def k4(qkvs_hbm, p0_hbm, p1_hbm, w1_hbm, w2_hbm, out_hbm,
           buf_a, buf_b, i0_v, i1_v, w1_v, w2_v, sem):
        wid = lax.axis_index("s") * 2 + lax.axis_index("c")

        def round_body(r, carry):
            pltpu.sync_copy(p0_hbm.at[wid, r], i0_v)
            pltpu.sync_copy(p1_hbm.at[wid, r], i1_v)
            pltpu.sync_copy(w1_hbm.at[wid, r], w1_v)
            pltpu.sync_copy(w2_hbm.at[wid, r], w2_v)
            pltpu.async_copy(qkvs_hbm.at[i0_v], buf_a, sem).wait()
            pltpu.async_copy(qkvs_hbm.at[i1_v], buf_b, sem).wait()

            def row_body(i, c2):
                wa = w1_v[i, 0, :]
                wb = w2_v[i, 0, :]
                for j in range(nmaj):
                    for k in range(nsub):
                        sl = pl.ds(k * 16, 16)
                        buf_a[i, j, sl] = (buf_a[i, j, sl] * wa
                                           + buf_b[i, j, sl] * wb)
                return c2

            lax.fori_loop(0, CB, row_body, 0)
            base = wid * TPW + r * CB
            pltpu.sync_copy(buf_a, out_hbm.at[pl.ds(base, CB)])
            return carry

        lax.fori_loop(0, rounds, round_body, 0)

    return k4(qkvs, pos0, pos1, w1, w2)


def _attn_kernel(qkv_ref, wp_ref, bp_ref, out_ref):
    bt = qkv_ref.shape[0]
    q3 = qkv_ref[:, :DIM].reshape(bt, NUM_HEADS, HEAD_DIM)
    k3 = qkv_ref[:, DIM:2 * DIM].reshape(bt, NUM_HEADS, HEAD_DIM)
    v3 = qkv_ref[:, 2 * DIM:].reshape(bt, NUM_HEADS, HEAD_DIM)
    attn = jax.lax.dot_general(
        q3, k3, (((2,), (2,)), ((0,), (0,))),
        preferred_element_type=jnp.float32) * SCALE
    attn = attn - jnp.max(attn, axis=2, keepdims=True)
    attn = jnp.exp(attn)
    attn = (attn / jnp.sum(attn, axis=2, keepdims=True)).astype(v3.dtype)
    ctx = jax.lax.dot_general(
        attn, v3, (((2,), (1,)), ((0,), (0,))),
        preferred_element_type=jnp.float32)
    ctx = ctx.reshape(bt, DIM).astype(jnp.bfloat16)
    out_ref[...] = jnp.dot(ctx, wp_ref[...],
                           preferred_element_type=jnp.float32) + bp_ref[...]


def _run_attn(qkv, W_proj_perm, b_proj):
    return pl.pallas_call(
        _attn_kernel,
        grid=(B // BT_ATTN,),
        in_specs=[
            pl.BlockSpec((BT_ATTN, 3 * DIM), lambda t: (t, 0)),
            pl.BlockSpec((DIM, DIM), lambda t: (0, 0)),
            pl.BlockSpec((1, DIM), lambda t: (0, 0)),
        ],
        out_specs=pl.BlockSpec((BT_ATTN, DIM), lambda t: (t, 0)),
        out_shape=jax.ShapeDtypeStruct((B, DIM), jnp.float32),
        compiler_params=pltpu.CompilerParams(
            dimension_semantics=("arbitrary",),
        ),
    )(qkv, W_proj_perm, b_proj.reshape(1, DIM))


@jax.jit
def kernel(x, y, W_qkv, W_gate, b_gate, W_proj, b_proj):
    Wp = (W_proj.reshape(HEAD_DIM, NUM_HEADS, DIM)
          .transpose(1, 0, 2).reshape(DIM, DIM).astype(jnp.bfloat16))

    pos0, pos1, w1, w2, eb = _run_router(x, W_gate, b_gate)
    hb = TPW // 2
    rounds = TPW // CB
    xs, ys = _dispatch_sc(x, y,
                          pos0.reshape(NW, 2, hb), pos1.reshape(NW, 2, hb))
    qkvs = _run_gemm(xs, ys, W_qkv, eb.reshape(128))
    qkv = _combine_sc(qkvs.reshape(PADDED, 3 * DIM // 128, 128),
                      pos0.reshape(NW, rounds, CB),
                      pos1.reshape(NW, rounds, CB),
                      w1.reshape(NW, rounds, CB, 1, 16),
                      w2.reshape(NW, rounds, CB, 1, 16))
    return _run_attn(qkv.reshape(B, 3 * DIM), Wp, b_proj)


# speedup vs baseline: 1.7915x; 1.7915x over previous
import jax
import jax.numpy as jnp
from jax.experimental import pallas as pl
from jax.experimental.pallas import tpu as pltpu

B = 4096
DIM = 1024
NUM_EXPERTS = 8
NUM_HEADS = 16
TOP_K = 2
HEAD_DIM = DIM // NUM_HEADS
SCALE = HEAD_DIM ** (-0.5)

BT = 512


def _routing_weights(scores):
    bt = scores.shape[0]
    e_iota = jax.lax.broadcasted_iota(jnp.int32, (bt, NUM_EXPERTS), 1)
    m1 = jnp.max(scores, axis=1, keepdims=True)
    idx1 = jnp.min(jnp.where(scores == m1, e_iota, NUM_EXPERTS), axis=1,
                   keepdims=True)
    masked = jnp.where(e_iota == idx1, -1.0, scores)
    m2 = jnp.max(masked, axis=1, keepdims=True)
    idx2 = jnp.min(jnp.where(masked == m2, e_iota, NUM_EXPERTS), axis=1,
                   keepdims=True)
    return jnp.where(e_iota == idx1, m1, 0.0) + jnp.where(e_iota == idx2, m2, 0.0)


def _attention(q, kv, wproj, bproj):
    bt = q.shape[0]
    q3 = q.reshape(bt, NUM_HEADS, HEAD_DIM)
    k3 = kv[:, :DIM].reshape(bt, NUM_HEADS, HEAD_DIM)
    v3 = kv[:, DIM:].reshape(bt, NUM_HEADS, HEAD_DIM)
    attn = jax.lax.dot_general(
        q3, k3, (((2,), (2,)), ((0,), (0,))),
        preferred_element_type=jnp.float32) * SCALE
    attn = attn - jnp.max(attn, axis=2, keepdims=True)
    attn = jnp.exp(attn)
    attn = attn / jnp.sum(attn, axis=2, keepdims=True)
    ctx = jax.lax.dot_general(
        attn, v3, (((2,), (1,)), ((0,), (0,))),
        preferred_element_type=jnp.float32)
    ctx = ctx.reshape(bt, DIM)
    return jnp.dot(ctx, wproj, preferred_element_type=jnp.float32) + bproj


def _moe_kernel(x_ref, y_ref, w_ref, wg_ref, bg_ref, wp_ref, bp_ref,
                out_ref, accq_ref, acckv_ref, gates_ref):
    e = pl.program_id(1)

    @pl.when(e == 0)
    def _():
        scores = jnp.dot(x_ref[...], wg_ref[...],
                         preferred_element_type=jnp.float32) + bg_ref[...]
        scores = scores - jnp.max(scores, axis=1, keepdims=True)
        scores = jnp.exp(scores)
        scores = scores / jnp.sum(scores, axis=1, keepdims=True)
        gates_ref[...] = _routing_weights(scores)

    gates = gates_ref[...]
    lane = jax.lax.broadcasted_iota(jnp.int32, gates.shape, 1)
    we = jnp.sum(jnp.where(lane == e, gates, 0.0), axis=1, keepdims=True)
    wq = w_ref[0, :, :DIM]
    wkv = w_ref[0, :, DIM:]
    contrib_q = we * jnp.dot(y_ref[...], wq, preferred_element_type=jnp.float32)
    contrib_kv = we * jnp.dot(x_ref[...], wkv, preferred_element_type=jnp.float32)

    @pl.when(e == 0)
    def _():
        accq_ref[...] = contrib_q
        acckv_ref[...] = contrib_kv

    @pl.when(e > 0)
    def _():
        accq_ref[...] += contrib_q
        acckv_ref[...] += contrib_kv

    @pl.when(e == NUM_EXPERTS - 1)
    def _():
        sub = 256
        for s in range(BT // sub):
            lo = s * sub
            out_ref[lo:lo + sub, :] = _attention(
                accq_ref[lo:lo + sub, :], acckv_ref[lo:lo + sub, :],
                wp_ref[...], bp_ref[...])


@jax.jit
def kernel(x, y, W_qkv, W_gate, b_gate, W_proj, b_proj):
    nt = B // BT
    W_proj_perm = (W_proj.reshape(HEAD_DIM, NUM_HEADS, DIM)
                   .transpose(1, 0, 2).reshape(DIM, DIM))
    out = pl.pallas_call(
        _moe_kernel,
        grid=(nt, NUM_EXPERTS),
        in_specs=[
            pl.BlockSpec((BT, DIM), lambda t, e: (t, 0)),
            pl.BlockSpec((BT, DIM), lambda t, e: (t, 0)),
            pl.BlockSpec((1, DIM, 3 * DIM), lambda t, e: (e, 0, 0)),
            pl.BlockSpec((DIM, NUM_EXPERTS), lambda t, e: (0, 0)),
            pl.BlockSpec((1, NUM_EXPERTS), lambda t, e: (0, 0)),
            pl.BlockSpec((DIM, DIM), lambda t, e: (0, 0)),
            pl.BlockSpec((1, DIM), lambda t, e: (0, 0)),
        ],
        out_specs=pl.BlockSpec((BT, DIM), lambda t, e: (t, 0)),
        out_shape=jax.ShapeDtypeStruct((B, DIM), jnp.float32),
        scratch_shapes=[
            pltpu.VMEM((BT, DIM), jnp.float32),
            pltpu.VMEM((BT, 2 * DIM), jnp.float32),
            pltpu.VMEM((BT, NUM_EXPERTS), jnp.float32),
        ],
        compiler_params=pltpu.CompilerParams(
            dimension_semantics=("arbitrary", "arbitrary"),
        ),
    )(x, y, W_qkv, W_gate, b_gate.reshape(1, NUM_EXPERTS),
      W_proj_perm, b_proj.reshape(1, DIM))
    return out


# speedup vs baseline: 1.8204x; 1.0161x over previous
import jax
import jax.numpy as jnp
from jax.experimental import pallas as pl
from jax.experimental.pallas import tpu as pltpu

B = 4096
DIM = 1024
NUM_EXPERTS = 8
NUM_HEADS = 16
TOP_K = 2
HEAD_DIM = DIM // NUM_HEADS
SCALE = HEAD_DIM ** (-0.5)

BT = 512


def _routing_weights(scores):
    bt = scores.shape[0]
    e_iota = jax.lax.broadcasted_iota(jnp.int32, (bt, NUM_EXPERTS), 1)
    m1 = jnp.max(scores, axis=1, keepdims=True)
    idx1 = jnp.min(jnp.where(scores == m1, e_iota, NUM_EXPERTS), axis=1,
                   keepdims=True)
    masked = jnp.where(e_iota == idx1, -1.0, scores)
    m2 = jnp.max(masked, axis=1, keepdims=True)
    idx2 = jnp.min(jnp.where(masked == m2, e_iota, NUM_EXPERTS), axis=1,
                   keepdims=True)
    return jnp.where(e_iota == idx1, m1, 0.0) + jnp.where(e_iota == idx2, m2, 0.0)


def _attention(q, kv, wproj, bproj):
    bt = q.shape[0]
    q3 = q.astype(jnp.bfloat16).reshape(bt, NUM_HEADS, HEAD_DIM)
    kvb = kv.astype(jnp.bfloat16)
    k3 = kvb[:, :DIM].reshape(bt, NUM_HEADS, HEAD_DIM)
    v3 = kvb[:, DIM:].reshape(bt, NUM_HEADS, HEAD_DIM)
    attn = jax.lax.dot_general(
        q3, k3, (((2,), (2,)), ((0,), (0,))),
        preferred_element_type=jnp.float32) * SCALE
    attn = attn - jnp.max(attn, axis=2, keepdims=True)
    attn = jnp.exp(attn)
    attn = (attn / jnp.sum(attn, axis=2, keepdims=True)).astype(jnp.bfloat16)
    ctx = jax.lax.dot_general(
        attn, v3, (((2,), (1,)), ((0,), (0,))),
        preferred_element_type=jnp.float32)
    ctx = ctx.reshape(bt, DIM).astype(jnp.bfloat16)
    return jnp.dot(ctx, wproj.astype(jnp.bfloat16),
                   preferred_element_type=jnp.float32) + bproj


def _moe_kernel(x_ref, y_ref, w_ref, wg_ref, bg_ref, wp_ref, bp_ref,
                out_ref, accq_ref, acckv_ref, gates_ref):
    e = pl.program_id(1)

    @pl.when(e == 0)
    def _():
        scores = jnp.dot(x_ref[...], wg_ref[...],
                         preferred_element_type=jnp.float32) + bg_ref[...]
        scores = scores - jnp.max(scores, axis=1, keepdims=True)
        scores = jnp.exp(scores)
        scores = scores / jnp.sum(scores, axis=1, keepdims=True)
        gates_ref[...] = _routing_weights(scores)

    gates = gates_ref[...]
    lane = jax.lax.broadcasted_iota(jnp.int32, gates.shape, 1)
    we = jnp.sum(jnp.where(lane == e, gates, 0.0), axis=1, keepdims=True)
    wq = w_ref[0, :, :DIM]
    wkv = w_ref[0, :, DIM:]
    contrib_q = we * jnp.dot(y_ref[...], wq, preferred_element_type=jnp.float32)
    contrib_kv = we * jnp.dot(x_ref[...], wkv, preferred_element_type=jnp.float32)

    @pl.when(e == 0)
    def _():
        accq_ref[...] = contrib_q
        acckv_ref[...] = contrib_kv

    @pl.when(e > 0)
    def _():
        accq_ref[...] += contrib_q
        acckv_ref[...] += contrib_kv

    @pl.when(e == NUM_EXPERTS - 1)
    def _():
        sub = 256
        for s in range(BT // sub):
            lo = s * sub
            out_ref[lo:lo + sub, :] = _attention(
                accq_ref[lo:lo + sub, :], acckv_ref[lo:lo + sub, :],
                wp_ref[...], bp_ref[...])


@jax.jit
def kernel(x, y, W_qkv, W_gate, b_gate, W_proj, b_proj):
    nt = B // BT
    W_proj_perm = (W_proj.reshape(HEAD_DIM, NUM_HEADS, DIM)
                   .transpose(1, 0, 2).reshape(DIM, DIM))
    out = pl.pallas_call(
        _moe_kernel,
        grid=(nt, NUM_EXPERTS),
        in_specs=[
            pl.BlockSpec((BT, DIM), lambda t, e: (t, 0)),
            pl.BlockSpec((BT, DIM), lambda t, e: (t, 0)),
            pl.BlockSpec((1, DIM, 3 * DIM), lambda t, e: (e, 0, 0)),
            pl.BlockSpec((DIM, NUM_EXPERTS), lambda t, e: (0, 0)),
            pl.BlockSpec((1, NUM_EXPERTS), lambda t, e: (0, 0)),
            pl.BlockSpec((DIM, DIM), lambda t, e: (0, 0)),
            pl.BlockSpec((1, DIM), lambda t, e: (0, 0)),
        ],
        out_specs=pl.BlockSpec((BT, DIM), lambda t, e: (t, 0)),
        out_shape=jax.ShapeDtypeStruct((B, DIM), jnp.float32),
        scratch_shapes=[
            pltpu.VMEM((BT, DIM), jnp.float32),
            pltpu.VMEM((BT, 2 * DIM), jnp.float32),
            pltpu.VMEM((BT, NUM_EXPERTS), jnp.float32),
        ],
        compiler_params=pltpu.CompilerParams(
            dimension_semantics=("arbitrary", "arbitrary"),
        ),
    )(x, y, W_qkv, W_gate, b_gate.reshape(1, NUM_EXPERTS),
      W_proj_perm, b_proj.reshape(1, DIM))
    return out


# speedup vs baseline: 1.8252x; 1.0026x over previous
import jax
import jax.numpy as jnp
from jax.experimental import pallas as pl
from jax.experimental.pallas import tpu as pltpu

B = 4096
DIM = 1024
NUM_EXPERTS = 8
NUM_HEADS = 16
TOP_K = 2
HEAD_DIM = DIM // NUM_HEADS
SCALE = HEAD_DIM ** (-0.5)

BT = 512


def _routing_weights(scores):
    bt = scores.shape[0]
    e_iota = jax.lax.broadcasted_iota(jnp.int32, (bt, NUM_EXPERTS), 1)
    m1 = jnp.max(scores, axis=1, keepdims=True)
    idx1 = jnp.min(jnp.where(scores == m1, e_iota, NUM_EXPERTS), axis=1,
                   keepdims=True)
    masked = jnp.where(e_iota == idx1, -1.0, scores)
    m2 = jnp.max(masked, axis=1, keepdims=True)
    idx2 = jnp.min(jnp.where(masked == m2, e_iota, NUM_EXPERTS), axis=1,
                   keepdims=True)
    return jnp.where(e_iota == idx1, m1, 0.0) + jnp.where(e_iota == idx2, m2, 0.0)


def _attention(q, kv, wproj, bproj):
    bt = q.shape[0]
    q3 = q.astype(jnp.bfloat16).reshape(bt, NUM_HEADS, HEAD_DIM)
    kvb = kv.astype(jnp.bfloat16)
    k3 = kvb[:, :DIM].reshape(bt, NUM_HEADS, HEAD_DIM)
    v3 = kvb[:, DIM:].reshape(bt, NUM_HEADS, HEAD_DIM)
    attn = jax.lax.dot_general(
        q3, k3, (((2,), (2,)), ((0,), (0,))),
        preferred_element_type=jnp.float32) * SCALE
    attn = attn - jnp.max(attn, axis=2, keepdims=True)
    attn = jnp.exp(attn)
    attn = (attn / jnp.sum(attn, axis=2, keepdims=True)).astype(jnp.bfloat16)
    ctx = jax.lax.dot_general(
        attn, v3, (((2,), (1,)), ((0,), (0,))),
        preferred_element_type=jnp.float32)
    ctx = ctx.reshape(bt, DIM).astype(jnp.bfloat16)
    return jnp.dot(ctx, wproj.astype(jnp.bfloat16),
                   preferred_element_type=jnp.float32) + bproj


def _moe_kernel(x_ref, y_ref, w_ref, wg_ref, bg_ref, wp_ref, bp_ref,
                out_ref, accq_ref, acckv_ref, gates_ref):
    e = pl.program_id(1)

    @pl.when(e == 0)
    def _():
        scores = jnp.dot(x_ref[...], wg_ref[...],
                         preferred_element_type=jnp.float32) + bg_ref[...]
        scores = scores - jnp.max(scores, axis=1, keepdims=True)
        scores = jnp.exp(scores)
        scores = scores / jnp.sum(scores, axis=1, keepdims=True)
        gates_ref[...] = _routing_weights(scores)

    gates = gates_ref[...]
    lane = jax.lax.broadcasted_iota(jnp.int32, gates.shape, 1)
    we = jnp.sum(jnp.where(lane == e, gates, 0.0), axis=1, keepdims=True)
    wq = w_ref[0, :, :DIM]
    wkv = w_ref[0, :, DIM:]
    contrib_q = we * jnp.dot(y_ref[...], wq, preferred_element_type=jnp.float32)
    contrib_kv = we * jnp.dot(x_ref[...], wkv, preferred_element_type=jnp.float32)

    @pl.when(e == 0)
    def _():
        accq_ref[...] = contrib_q
        acckv_ref[...] = contrib_kv

    @pl.when(e > 0)
    def _():
        accq_ref[...] += contrib_q
        acckv_ref[...] += contrib_kv

    @pl.when(e == NUM_EXPERTS - 1)
    def _():
        sub = 512
        for s in range(BT // sub):
            lo = s * sub
            out_ref[lo:lo + sub, :] = _attention(
                accq_ref[lo:lo + sub, :], acckv_ref[lo:lo + sub, :],
                wp_ref[...], bp_ref[...])


@jax.jit
def kernel(x, y, W_qkv, W_gate, b_gate, W_proj, b_proj):
    nt = B // BT
    W_proj_perm = (W_proj.reshape(HEAD_DIM, NUM_HEADS, DIM)
                   .transpose(1, 0, 2).reshape(DIM, DIM))
    out = pl.pallas_call(
        _moe_kernel,
        grid=(nt, NUM_EXPERTS),
        in_specs=[
            pl.BlockSpec((BT, DIM), lambda t, e: (t, 0)),
            pl.BlockSpec((BT, DIM), lambda t, e: (t, 0)),
            pl.BlockSpec((1, DIM, 3 * DIM), lambda t, e: (e, 0, 0)),
            pl.BlockSpec((DIM, NUM_EXPERTS), lambda t, e: (0, 0)),
            pl.BlockSpec((1, NUM_EXPERTS), lambda t, e: (0, 0)),
            pl.BlockSpec((DIM, DIM), lambda t, e: (0, 0)),
            pl.BlockSpec((1, DIM), lambda t, e: (0, 0)),
        ],
        out_specs=pl.BlockSpec((BT, DIM), lambda t, e: (t, 0)),
        out_shape=jax.ShapeDtypeStruct((B, DIM), jnp.float32),
        scratch_shapes=[
            pltpu.VMEM((BT, DIM), jnp.float32),
            pltpu.VMEM((BT, 2 * DIM), jnp.float32),
            pltpu.VMEM((BT, NUM_EXPERTS), jnp.float32),
        ],
        compiler_params=pltpu.CompilerParams(
            dimension_semantics=("arbitrary", "arbitrary"),
        ),
    )(x, y, W_qkv, W_gate, b_gate.reshape(1, NUM_EXPERTS),
      W_proj_perm, b_proj.reshape(1, DIM))
    return out
